# baseline (device time: 45004 ns/iter reference)
import jax
import jax.numpy as jnp
from jax import lax
from jax.experimental import pallas as pl
from jax.experimental.pallas import tpu as pltpu

B, H, D, BS = 16, 16, 64, 16
NB = 128
PAGES = 128
HD = H * D
NK = PAGES * BS
BH = B * H
BNB = B * NB
SCALE = D ** -0.5
NEG = -1e30


def _iota2(shape, dim):
    return lax.broadcasted_iota(jnp.int32, shape, dim)


def _body(q_ref, k_ref, v_ref, bt_ref, lens_ref, out_ref,
          o_send, o_recv, m_send, m_recv, l_send, l_recv,
          send_sems, recv_sems):
    my_x = lax.axis_index("x")
    my_y = lax.axis_index("y")
    my_z = lax.axis_index("z")
    nbr = (my_x, 1 - my_y, my_z)

    barrier = pltpu.get_barrier_semaphore()
    pl.semaphore_signal(barrier, inc=1, device_id=nbr,
                        device_id_type=pl.DeviceIdType.MESH)
    pl.semaphore_wait(barrier, 1)

    maskB_bf = (_iota2((BH, HD), 1) // D == _iota2((BH, HD), 0) % H
                ).astype(jnp.bfloat16)
    erep_bf = (_iota2((BH, B), 0) // H == _iota2((BH, B), 1)
               ).astype(jnp.bfloat16)
    gsum_bf = (_iota2((BNB, B), 0) // NB == _iota2((BNB, B), 1)
               ).astype(jnp.bfloat16)
    erow_bf = (_iota2((NK, PAGES), 0) // BS == _iota2((NK, PAGES), 1)
               ).astype(jnp.bfloat16)
    gsum_f32 = (_iota2((BNB, B), 0) // NB == _iota2((BNB, B), 1)
                ).astype(jnp.float32)

    kb = k_ref[...].reshape(NK, HD)
    vb = v_ref[...].reshape(NK, HD)
    qr = q_ref[...].reshape(B, HD)

    lensf = lens_ref[...].astype(jnp.float32)
    lensflat = lax.dot_general(lensf, gsum_f32, (((1,), (1,)), ((), ())),
                               preferred_element_type=jnp.float32)
    jflat = (_iota2((1, BNB), 1) % NB).astype(jnp.float32)
    valid = jflat < lensflat
    btlocal = bt_ref[...] - my_y * PAGES
    pkp = _iota2((PAGES, BNB), 0)
    match = jnp.logical_and(pkp == btlocal, valid).astype(jnp.bfloat16)
    cnt_pages = lax.dot_general(match, gsum_bf, (((1,), (0,)), ((), ())),
                                preferred_element_type=jnp.float32)
    cnt_pb = lax.dot_general(cnt_pages.astype(jnp.bfloat16), erep_bf,
                             (((1,), (1,)), ((), ())),
                             preferred_element_type=jnp.float32)
    cntcol = lax.dot_general(erow_bf, cnt_pb.astype(jnp.bfloat16),
                             (((1,), (0,)), ((), ())),
                             preferred_element_type=jnp.float32)

    qrep = lax.dot_general(erep_bf, qr, (((1,), (0,)), ((), ())),
                           preferred_element_type=jnp.float32)
    qbT = qrep.astype(jnp.bfloat16) * maskB_bf

    s = lax.dot_general(kb, qbT, (((1,), (1,)), ((), ())),
                        preferred_element_type=jnp.float32) * SCALE

    smask = jnp.where(cntcol > 0, s, NEG)
    m = jnp.max(smask, axis=0, keepdims=True)
    m_safe = jnp.where(m < -1e29, 0.0, m)
    p = jnp.exp(s - m_safe) * cntcol
    l = jnp.sum(p, axis=0, keepdims=True)

    r = lax.dot_general(p.astype(jnp.bfloat16), vb,
                        (((0,), (0,)), ((), ())),
                        preferred_element_type=jnp.float32)
    okh = jnp.zeros((BH, D), jnp.float32)
    for j in range(H):
        colmask = (_iota2((BH, 1), 0) % H == j).astype(jnp.float32)
        okh = okh + r[:, j * D:(j + 1) * D] * colmask
    o_send[...] = okh
    m_send[...] = m
    l_send[...] = l

    rdmas = []
    for i, (src, dst) in enumerate(
            [(o_send, o_recv), (m_send, m_recv), (l_send, l_recv)]):
        rdma = pltpu.make_async_remote_copy(
            src_ref=src, dst_ref=dst,
            send_sem=send_sems.at[i], recv_sem=recv_sems.at[i],
            device_id=nbr, device_id_type=pl.DeviceIdType.MESH)
        rdma.start()
        rdmas.append(rdma)
    for rdma in rdmas:
        rdma.wait()

    m_loc = m_send[...]
    m_rem = m_recv[...]
    mt = jnp.maximum(m_loc, m_rem)
    a = jnp.exp(m_loc - mt)
    c = jnp.exp(m_rem - mt)
    lt = a * l_send[...] + c * l_recv[...]
    eye_f32 = (_iota2((BH, BH), 0) == _iota2((BH, BH), 1)).astype(jnp.float32)
    tr = (((1,), (1,)), ((), ()))
    a_col = lax.dot_general(eye_f32, a, tr, preferred_element_type=jnp.float32)
    c_col = lax.dot_general(eye_f32, c, tr, preferred_element_type=jnp.float32)
    l_col = lax.dot_general(eye_f32, lt, tr, preferred_element_type=jnp.float32)
    out_ref[...] = (a_col * o_send[...] + c_col * o_recv[...]) / l_col


def kernel(Q, K, V, bt, lens):
    Qr = Q.astype(jnp.bfloat16)
    Kr = K.astype(jnp.bfloat16)
    Vr = V.astype(jnp.bfloat16)
    btr = bt.reshape(1, BNB)
    lensr = lens.reshape(1, B)

    out = pl.pallas_call(
        _body,
        out_shape=jax.ShapeDtypeStruct((BH, D), jnp.float32),
        in_specs=[
            pl.BlockSpec(memory_space=pltpu.VMEM),
            pl.BlockSpec(memory_space=pltpu.VMEM),
            pl.BlockSpec(memory_space=pltpu.VMEM),
            pl.BlockSpec(memory_space=pltpu.VMEM),
            pl.BlockSpec(memory_space=pltpu.VMEM),
        ],
        out_specs=pl.BlockSpec(memory_space=pltpu.VMEM),
        scratch_shapes=[
            pltpu.VMEM((BH, D), jnp.float32),
            pltpu.VMEM((BH, D), jnp.float32),
            pltpu.VMEM((1, BH), jnp.float32),
            pltpu.VMEM((1, BH), jnp.float32),
            pltpu.VMEM((1, BH), jnp.float32),
            pltpu.VMEM((1, BH), jnp.float32),
            pltpu.SemaphoreType.DMA((3,)),
            pltpu.SemaphoreType.DMA((3,)),
        ],
        compiler_params=pltpu.CompilerParams(collective_id=0),
    )(Qr, Kr, Vr, btr, lensr)
    return out.reshape(B, 1, H, D)


# device time: 21876 ns/iter; 2.0572x vs baseline; 2.0572x over previous
import jax
import jax.numpy as jnp
from jax import lax
from jax.experimental import pallas as pl
from jax.experimental.pallas import tpu as pltpu

B, H, D, BS = 16, 16, 64, 16
NB = 128
PAGES = 128
HD = H * D
BH = B * H
BNB = B * NB
THD = BS * HD
SCALE = D ** -0.5
NEG = -1e30


def _iota2(shape, dim):
    return lax.broadcasted_iota(jnp.int32, shape, dim)


def _body(q_ref, kt_ref, vt_ref, bt_ref, lens_ref, out_ref,
          o_send, o_recv, m_send, m_recv, l_send, l_recv,
          send_sems, recv_sems):
    my_x = lax.axis_index("x")
    my_y = lax.axis_index("y")
    my_z = lax.axis_index("z")
    nbr = (my_x, 1 - my_y, my_z)

    barrier = pltpu.get_barrier_semaphore()
    pl.semaphore_signal(barrier, inc=1, device_id=nbr,
                        device_id_type=pl.DeviceIdType.MESH)
    pl.semaphore_wait(barrier, 1)

    maskB_bf = (_iota2((BH, HD), 1) // D == _iota2((BH, HD), 0) % H
                ).astype(jnp.bfloat16)
    erep_bf = (_iota2((BH, B), 0) // H == _iota2((BH, B), 1)
               ).astype(jnp.bfloat16)
    gsum_bf = (_iota2((BNB, B), 0) // NB == _iota2((BNB, B), 1)
               ).astype(jnp.bfloat16)
    gsum_f32 = (_iota2((BNB, B), 0) // NB == _iota2((BNB, B), 1)
                ).astype(jnp.float32)

    lensf = lens_ref[...].astype(jnp.float32)
    lensflat = lax.dot_general(lensf, gsum_f32, (((1,), (1,)), ((), ())),
                               preferred_element_type=jnp.float32)
    jflat = (_iota2((1, BNB), 1) % NB).astype(jnp.float32)
    valid = jflat < lensflat
    btlocal = bt_ref[...] - my_y * PAGES
    pkp = _iota2((PAGES, BNB), 0)
    match = jnp.logical_and(pkp == btlocal, valid).astype(jnp.bfloat16)
    cnt_pages = lax.dot_general(match, gsum_bf, (((1,), (0,)), ((), ())),
                                preferred_element_type=jnp.float32)
    cnt_bhp = lax.dot_general(erep_bf, cnt_pages.astype(jnp.bfloat16),
                              (((1,), (1,)), ((), ())),
                              preferred_element_type=jnp.float32)

    qr = q_ref[...].astype(jnp.bfloat16).reshape(B, HD)
    qrep = lax.dot_general(erep_bf, qr, (((1,), (0,)), ((), ())),
                           preferred_element_type=jnp.float32)
    qbT = qrep.astype(jnp.bfloat16) * maskB_bf

    ktb = kt_ref[...].astype(jnp.bfloat16)
    vtb = vt_ref[...].astype(jnp.bfloat16)

    s_ts = []
    mrun = jnp.full((BH, PAGES), NEG, jnp.float32)
    for t in range(BS):
        k_t = ktb[t * HD:(t + 1) * HD, :]
        s_t = lax.dot_general(qbT, k_t, (((1,), (0,)), ((), ())),
                              preferred_element_type=jnp.float32) * SCALE
        s_ts.append(s_t)
        mrun = jnp.maximum(mrun, jnp.where(cnt_bhp > 0, s_t, NEG))
    m = jnp.max(mrun, axis=1, keepdims=True)
    m_safe = jnp.where(m < -1e29, 0.0, m)

    l = jnp.zeros((BH, 1), jnp.float32)
    racc = jnp.zeros((BH, HD), jnp.float32)
    for t in range(BS):
        p_t = jnp.exp(s_ts[t] - m_safe) * cnt_bhp
        l = l + jnp.sum(p_t, axis=1, keepdims=True)
        v_t = vtb[t * HD:(t + 1) * HD, :]
        r_t = lax.dot_general(p_t.astype(jnp.bfloat16), v_t,
                              (((1,), (1,)), ((), ())),
                              preferred_element_type=jnp.float32)
        racc = racc + r_t

    okh = jnp.zeros((BH, D), jnp.float32)
    for j in range(H):
        colmask = (_iota2((BH, 1), 0) % H == j).astype(jnp.float32)
        okh = okh + racc[:, j * D:(j + 1) * D] * colmask
    o_send[...] = okh
    m_send[...] = m
    l_send[...] = l

    rdmas = []
    for i, (src, dst) in enumerate(
            [(o_send, o_recv), (m_send, m_recv), (l_send, l_recv)]):
        rdma = pltpu.make_async_remote_copy(
            src_ref=src, dst_ref=dst,
            send_sem=send_sems.at[i], recv_sem=recv_sems.at[i],
            device_id=nbr, device_id_type=pl.DeviceIdType.MESH)
        rdma.start()
        rdmas.append(rdma)
    for rdma in rdmas:
        rdma.wait()

    m_loc = m_send[...]
    m_rem = m_recv[...]
    mt = jnp.maximum(m_loc, m_rem)
    a = jnp.exp(m_loc - mt)
    c = jnp.exp(m_rem - mt)
    lt = a * l_send[...] + c * l_recv[...]
    out_ref[...] = (a * o_send[...] + c * o_recv[...]) / lt


def kernel(Q, K, V, bt, lens):
    Kt = jnp.transpose(K, (1, 2, 3, 0)).reshape(THD, PAGES)
    Vt = jnp.transpose(V, (1, 2, 3, 0)).reshape(THD, PAGES)
    btr = bt.reshape(1, BNB)
    lensr = lens.reshape(1, B)

    out = pl.pallas_call(
        _body,
        out_shape=jax.ShapeDtypeStruct((BH, D), jnp.float32),
        in_specs=[
            pl.BlockSpec(memory_space=pltpu.VMEM),
            pl.BlockSpec(memory_space=pltpu.VMEM),
            pl.BlockSpec(memory_space=pltpu.VMEM),
            pl.BlockSpec(memory_space=pltpu.VMEM),
            pl.BlockSpec(memory_space=pltpu.VMEM),
        ],
        out_specs=pl.BlockSpec(memory_space=pltpu.VMEM),
        scratch_shapes=[
            pltpu.VMEM((BH, D), jnp.float32),
            pltpu.VMEM((BH, D), jnp.float32),
            pltpu.VMEM((BH, 1), jnp.float32),
            pltpu.VMEM((BH, 1), jnp.float32),
            pltpu.VMEM((BH, 1), jnp.float32),
            pltpu.VMEM((BH, 1), jnp.float32),
            pltpu.SemaphoreType.DMA((3,)),
            pltpu.SemaphoreType.DMA((3,)),
        ],
        compiler_params=pltpu.CompilerParams(
            collective_id=0, vmem_limit_bytes=100 * 1024 * 1024),
    )(Q, Kt, Vt, btr, lensr)
    return out.reshape(B, 1, H, D)
